# Initial kernel scaffold; baseline (speedup 1.0000x reference)
import jax
import jax.numpy as jnp
from jax.experimental import pallas as pl


def kernel(x, edge_index, W1, att_s1, att_d1, b1, W2, att_s2, att_d2, b2, fc1_w, fc1_b, fc2_w, fc2_b):
    raise NotImplementedError("write your pallas kernel here")


# speedup vs baseline: 1.6347x; 1.6347x over previous
import functools

import jax
import jax.numpy as jnp
from jax.experimental import pallas as pl
from jax.experimental.pallas import tpu as pltpu

N = 10000
D = 128
OUT = 64
ROWS = 1000


def _head_body(h_ref, fc1w_ref, fc1b_ref, fc2w_ref, fc2b_ref, out_ref):
    h = h_ref[...]
    z = jnp.maximum(h @ fc1w_ref[...] + fc1b_ref[...][None, :], 0.0)
    y = z @ fc2w_ref[...] + fc2b_ref[...][None, :]
    y = y - jnp.max(y, axis=1, keepdims=True)
    e = jnp.exp(y)
    out_ref[...] = e / jnp.sum(e, axis=1, keepdims=True)


def _mlp_head(h, fc1_w, fc1_b, fc2_w, fc2_b):
    grid = (N // ROWS,)
    return pl.pallas_call(
        _head_body,
        grid=grid,
        in_specs=[
            pl.BlockSpec((ROWS, D), lambda i: (i, 0)),
            pl.BlockSpec((D, D), lambda i: (0, 0)),
            pl.BlockSpec((D,), lambda i: (0,)),
            pl.BlockSpec((D, OUT), lambda i: (0, 0)),
            pl.BlockSpec((OUT,), lambda i: (0,)),
        ],
        out_specs=pl.BlockSpec((ROWS, OUT), lambda i: (i, 0)),
        out_shape=jax.ShapeDtypeStruct((N, OUT), jnp.float32),
    )(h, fc1_w, fc1_b, fc2_w, fc2_b)


def _gat_layer(x, W, att_s, att_d, b, edge_index):
    src = edge_index[0]
    dst = edge_index[1]
    h = x @ W
    a_src = h @ att_s[0]
    a_dst = h @ att_d[0]
    e = a_src[src] + a_dst[dst]
    e = jnp.where(e >= 0, e, 0.2 * e)
    ee = jnp.exp(e)
    den = jax.ops.segment_sum(ee, dst, num_segments=N)
    agg = jax.ops.segment_sum(h[src] * ee[:, None], dst, num_segments=N)
    es = a_src + a_dst
    es = jnp.where(es >= 0, es, 0.2 * es)
    ws = jnp.exp(es)
    den = den + ws
    agg = agg + ws[:, None] * h
    return agg / (den[:, None] + 1e-16) + b


def kernel(x, edge_index, W1, att_s1, att_d1, b1, W2, att_s2, att_d2, b2,
           fc1_w, fc1_b, fc2_w, fc2_b):
    h = jnp.maximum(_gat_layer(x, W1, att_s1, att_d1, b1, edge_index), 0.0)
    h = jnp.maximum(_gat_layer(h, W2, att_s2, att_d2, b2, edge_index), 0.0)
    return _mlp_head(h, fc1_w, fc1_b, fc2_w, fc2_b)


# speedup vs baseline: 20.2455x; 12.3846x over previous
import functools

import jax
import jax.numpy as jnp
from jax import lax
from jax.experimental import pallas as pl
from jax.experimental.pallas import tpu as pltpu
from jax.experimental.pallas import tpu_sc as plsc

N = 10000
D = 128
OUT = 64
ROWS = 1000
NW = 32
EPW = N
CHUNK = 128
NCHUNK = 79
EPAD = NCHUNK * CHUNK
NSUB = 16
DENW = EPAD // NSUB


def _leaky(e):
    return jnp.where(e >= 0.0, e, 0.2 * e)


def _pre_body(x_ref, w_ref, atts_ref, attd_ref, h_ref, as_ref, ad_ref, ws_ref):
    h = x_ref[...] @ w_ref[...]
    a_s = jnp.sum(h * atts_ref[...], axis=1, keepdims=True)
    a_d = jnp.sum(h * attd_ref[...], axis=1, keepdims=True)
    h_ref[...] = h
    as_ref[...] = a_s
    ad_ref[...] = a_d
    ws_ref[...] = jnp.exp(_leaky(a_s + a_d))


def _tc_pre(x, W, att_s, att_d):
    return pl.pallas_call(
        _pre_body,
        grid=(N // ROWS,),
        in_specs=[
            pl.BlockSpec((ROWS, D), lambda i: (i, 0)),
            pl.BlockSpec((D, D), lambda i: (0, 0)),
            pl.BlockSpec((1, D), lambda i: (0, 0)),
            pl.BlockSpec((1, D), lambda i: (0, 0)),
        ],
        out_specs=[
            pl.BlockSpec((ROWS, D), lambda i: (i, 0)),
            pl.BlockSpec((ROWS, 1), lambda i: (i, 0)),
            pl.BlockSpec((ROWS, 1), lambda i: (i, 0)),
            pl.BlockSpec((ROWS, 1), lambda i: (i, 0)),
        ],
        out_shape=[
            jax.ShapeDtypeStruct((N, D), jnp.float32),
            jax.ShapeDtypeStruct((N, 1), jnp.float32),
            jax.ShapeDtypeStruct((N, 1), jnp.float32),
            jax.ShapeDtypeStruct((N, 1), jnp.float32),
        ],
    )(x, W, att_s, att_d)


def _norm_x(agg_ref, den_ref, ws_ref, hp_ref, b_ref):
    ws = ws_ref[...]
    agg = agg_ref[0] + agg_ref[1] + ws * hp_ref[...]
    den = (den_ref[0, 0] + den_ref[0, 1])[:, None] + ws + 1e-16
    return jnp.maximum(agg / den + b_ref[...], 0.0)


def _mid_body(agg_ref, den_ref, ws_ref, hp_ref, b_ref, w_ref, atts_ref,
              attd_ref, h_ref, as_ref, ad_ref, ws2_ref):
    x2 = _norm_x(agg_ref, den_ref, ws_ref, hp_ref, b_ref)
    h = x2 @ w_ref[...]
    a_s = jnp.sum(h * atts_ref[...], axis=1, keepdims=True)
    a_d = jnp.sum(h * attd_ref[...], axis=1, keepdims=True)
    h_ref[...] = h
    as_ref[...] = a_s
    ad_ref[...] = a_d
    ws2_ref[...] = jnp.exp(_leaky(a_s + a_d))


def _tc_mid(agg, den, ws, h_prev, b, W, att_s, att_d):
    return pl.pallas_call(
        _mid_body,
        grid=(N // ROWS,),
        in_specs=[
            pl.BlockSpec((2, ROWS, D), lambda i: (0, i, 0)),
            pl.BlockSpec((1, 2, ROWS), lambda i: (i, 0, 0)),
            pl.BlockSpec((ROWS, 1), lambda i: (i, 0)),
            pl.BlockSpec((ROWS, D), lambda i: (i, 0)),
            pl.BlockSpec((1, D), lambda i: (0, 0)),
            pl.BlockSpec((D, D), lambda i: (0, 0)),
            pl.BlockSpec((1, D), lambda i: (0, 0)),
            pl.BlockSpec((1, D), lambda i: (0, 0)),
        ],
        out_specs=[
            pl.BlockSpec((ROWS, D), lambda i: (i, 0)),
            pl.BlockSpec((ROWS, 1), lambda i: (i, 0)),
            pl.BlockSpec((ROWS, 1), lambda i: (i, 0)),
            pl.BlockSpec((ROWS, 1), lambda i: (i, 0)),
        ],
        out_shape=[
            jax.ShapeDtypeStruct((N, D), jnp.float32),
            jax.ShapeDtypeStruct((N, 1), jnp.float32),
            jax.ShapeDtypeStruct((N, 1), jnp.float32),
            jax.ShapeDtypeStruct((N, 1), jnp.float32),
        ],
    )(agg, den, ws, h_prev, b, W, att_s, att_d)


def _head_body(agg_ref, den_ref, ws_ref, hp_ref, b_ref, fc1w_ref, fc1b_ref,
               fc2w_ref, fc2b_ref, out_ref):
    x3 = _norm_x(agg_ref, den_ref, ws_ref, hp_ref, b_ref)
    z = jnp.maximum(x3 @ fc1w_ref[...] + fc1b_ref[...][None, :], 0.0)
    y = z @ fc2w_ref[...] + fc2b_ref[...][None, :]
    y = y - jnp.max(y, axis=1, keepdims=True)
    e = jnp.exp(y)
    out_ref[...] = e / jnp.sum(e, axis=1, keepdims=True)


def _tc_head(agg, den, ws, h_prev, b, fc1_w, fc1_b, fc2_w, fc2_b):
    return pl.pallas_call(
        _head_body,
        grid=(N // ROWS,),
        in_specs=[
            pl.BlockSpec((2, ROWS, D), lambda i: (0, i, 0)),
            pl.BlockSpec((1, 2, ROWS), lambda i: (i, 0, 0)),
            pl.BlockSpec((ROWS, 1), lambda i: (i, 0)),
            pl.BlockSpec((ROWS, D), lambda i: (i, 0)),
            pl.BlockSpec((1, D), lambda i: (0, 0)),
            pl.BlockSpec((D, D), lambda i: (0, 0)),
            pl.BlockSpec((D,), lambda i: (0,)),
            pl.BlockSpec((D, OUT), lambda i: (0, 0)),
            pl.BlockSpec((OUT,), lambda i: (0,)),
        ],
        out_specs=pl.BlockSpec((ROWS, OUT), lambda i: (i, 0)),
        out_shape=jax.ShapeDtypeStruct((N, OUT), jnp.float32),
    )(agg, den, ws, h_prev, b, fc1_w, fc1_b, fc2_w, fc2_b)


def _sc_body(h_hbm, as_hbm, ad_hbm, src_hbm, dst_hbm, agg_out, den_out,
             as_v, ad_v, src_b, dst_b, w_b, rows, agg_sh, den_sh, gsem):
    cid = lax.axis_index("c")
    sid = lax.axis_index("s")
    wid = sid * 2 + cid

    pltpu.sync_copy(as_hbm, as_v)
    pltpu.sync_copy(ad_hbm, ad_v)

    zeros = jnp.zeros((16,), jnp.float32)

    def zero_rows(i, _):
        r = i // (D // 16)
        q = i % (D // 16)
        rows[r, pl.ds(q * 16, 16)] = zeros
        return 0
    lax.fori_loop(0, CHUNK * (D // 16), zero_rows, 0)
    for q in range(8):
        w_b[pl.ds(q * 16, 16)] = zeros

    for q in range(4):
        pltpu.sync_copy(rows, agg_sh.at[pl.ds(sid * 625 + q * CHUNK, CHUNK)])
    pltpu.sync_copy(rows.at[pl.ds(0, 113)],
                    agg_sh.at[pl.ds(sid * 625 + 512, 113)])
    for q in range(4):
        pltpu.sync_copy(w_b, den_sh.at[pl.ds(sid * DENW + q * CHUNK, CHUNK)])
    pltpu.sync_copy(w_b.at[pl.ds(0, DENW - 512)],
                    den_sh.at[pl.ds(sid * DENW + 512, DENW - 512)])
    plsc.subcore_barrier()

    lane = lax.iota(jnp.int32, 16)

    def chunk(c, _):
        pltpu.sync_copy(src_hbm.at[wid, pl.ds(c * CHUNK, CHUNK)], src_b)
        pltpu.sync_copy(dst_hbm.at[wid, pl.ds(c * CHUNK, CHUNK)], dst_b)
        gd = pltpu.async_copy(h_hbm.at[src_b], rows, gsem)
        for g in range(CHUNK // 16):
            sv = src_b[pl.ds(g * 16, 16)]
            dv = dst_b[pl.ds(g * 16, 16)]
            e = plsc.load_gather(as_v, [sv]) + plsc.load_gather(ad_v, [dv])
            w = jnp.exp(_leaky(e))
            valid = (c * CHUNK + g * 16) + lane < EPW
            w_b[pl.ds(g * 16, 16)] = jnp.where(valid, w, 0.0)
        pltpu.sync_copy(w_b, den_sh.at[dst_b], add=True)
        gd.wait()

        def scale_row(i, _):
            wsp = plsc.load_gather(w_b, [jnp.full((16,), i, jnp.int32)])
            for j in range(D // 16):
                rows[i, pl.ds(j * 16, 16)] = rows[i, pl.ds(j * 16, 16)] * wsp
            return 0
        lax.fori_loop(0, CHUNK, scale_row, 0)

        pltpu.sync_copy(rows, agg_sh.at[dst_b], add=True)
        return 0

    lax.fori_loop(0, NCHUNK, chunk, 0)
    plsc.subcore_barrier()

    for q in range(4):
        pltpu.sync_copy(den_sh.at[pl.ds(sid * DENW + q * CHUNK, CHUNK)], w_b)
        pltpu.sync_copy(w_b, den_out.at[pl.ds(cid * EPAD + sid * DENW
                                              + q * CHUNK, CHUNK)])
    pltpu.sync_copy(den_sh.at[pl.ds(sid * DENW + 512, DENW - 512)],
                    w_b.at[pl.ds(0, DENW - 512)])
    pltpu.sync_copy(w_b.at[pl.ds(0, DENW - 512)],
                    den_out.at[pl.ds(cid * EPAD + sid * DENW + 512,
                                     DENW - 512)])
    pltpu.sync_copy(agg_sh.at[pl.ds(sid * 624, 624)],
                    agg_out.at[cid, pl.ds(sid * 624, 624)])

    @pl.when(sid == NSUB - 1)
    def _tail():
        pltpu.sync_copy(agg_sh.at[pl.ds(9984, 16)],
                        agg_out.at[cid, pl.ds(9984, 16)])


def _sc_layer(h, a_s, a_d, src3, dst3):
    mesh = plsc.VectorSubcoreMesh(core_axis_name="c", subcore_axis_name="s",
                                  num_cores=2, num_subcores=NSUB)
    f = pl.kernel(
        _sc_body,
        out_type=[
            jax.ShapeDtypeStruct((2, N, D), jnp.float32),
            jax.ShapeDtypeStruct((2 * EPAD,), jnp.float32),
        ],
        mesh=mesh,
        compiler_params=pltpu.CompilerParams(needs_layout_passes=False),
        scratch_types=[
            pltpu.VMEM((N,), jnp.float32),
            pltpu.VMEM((N,), jnp.float32),
            pltpu.VMEM((CHUNK,), jnp.int32),
            pltpu.VMEM((CHUNK,), jnp.int32),
            pltpu.VMEM((CHUNK,), jnp.float32),
            pltpu.VMEM((CHUNK, D), jnp.float32),
            pltpu.VMEM_SHARED((N, D), jnp.float32),
            pltpu.VMEM_SHARED((EPAD,), jnp.float32),
            pltpu.SemaphoreType.DMA,
        ],
    )
    return f(h, a_s, a_d, src3, dst3)


def kernel(x, edge_index, W1, att_s1, att_d1, b1, W2, att_s2, att_d2, b2,
           fc1_w, fc1_b, fc2_w, fc2_b):
    pad = jnp.zeros((NW, EPAD - EPW), jnp.int32)
    src3 = jnp.concatenate([edge_index[0].reshape(NW, EPW), pad], axis=1)
    dst3 = jnp.concatenate([edge_index[1].reshape(NW, EPW), pad], axis=1)
    b1r = b1.reshape(1, D)
    b2r = b2.reshape(1, D)

    def den_t(d):
        return (d.reshape(2, EPAD)[:, :N]
                .reshape(2, N // ROWS, ROWS).transpose(1, 0, 2))

    h1, as1, ad1, ws1 = _tc_pre(x, W1, att_s1, att_d1)
    agg1, den1 = _sc_layer(h1, as1.reshape(N), ad1.reshape(N), src3, dst3)
    h2, as2, ad2, ws2 = _tc_mid(agg1, den_t(den1), ws1, h1, b1r, W2,
                                att_s2, att_d2)
    agg2, den2 = _sc_layer(h2, as2.reshape(N), ad2.reshape(N), src3, dst3)
    return _tc_head(agg2, den_t(den2), ws2, h2, b2r, fc1_w, fc1_b,
                    fc2_w, fc2_b)


# speedup vs baseline: 25.9692x; 1.2827x over previous
import functools

import jax
import jax.numpy as jnp
from jax import lax
from jax.experimental import pallas as pl
from jax.experimental.pallas import tpu as pltpu
from jax.experimental.pallas import tpu_sc as plsc

N = 10000
D = 128
OUT = 64
ROWS = 1000
NW = 32
EPW = N
CHUNK = 128
NCHUNK = 79
EPAD = NCHUNK * CHUNK
NSUB = 16
DENW = EPAD // NSUB


def _leaky(e):
    return jnp.where(e >= 0.0, e, 0.2 * e)


def _pre_body(x_ref, w_ref, atts_ref, attd_ref, h_ref, as_ref, ad_ref, ws_ref):
    h = x_ref[...] @ w_ref[...]
    a_s = jnp.sum(h * atts_ref[...], axis=1, keepdims=True)
    a_d = jnp.sum(h * attd_ref[...], axis=1, keepdims=True)
    h_ref[...] = h
    as_ref[...] = a_s
    ad_ref[...] = a_d
    ws_ref[...] = jnp.exp(_leaky(a_s + a_d))


def _tc_pre(x, W, att_s, att_d):
    return pl.pallas_call(
        _pre_body,
        grid=(N // ROWS,),
        in_specs=[
            pl.BlockSpec((ROWS, D), lambda i: (i, 0)),
            pl.BlockSpec((D, D), lambda i: (0, 0)),
            pl.BlockSpec((1, D), lambda i: (0, 0)),
            pl.BlockSpec((1, D), lambda i: (0, 0)),
        ],
        out_specs=[
            pl.BlockSpec((ROWS, D), lambda i: (i, 0)),
            pl.BlockSpec((ROWS, 1), lambda i: (i, 0)),
            pl.BlockSpec((ROWS, 1), lambda i: (i, 0)),
            pl.BlockSpec((ROWS, 1), lambda i: (i, 0)),
        ],
        out_shape=[
            jax.ShapeDtypeStruct((N, D), jnp.float32),
            jax.ShapeDtypeStruct((N, 1), jnp.float32),
            jax.ShapeDtypeStruct((N, 1), jnp.float32),
            jax.ShapeDtypeStruct((N, 1), jnp.float32),
        ],
    )(x, W, att_s, att_d)


def _norm_x(agg_ref, den_ref, ws_ref, hp_ref, b_ref):
    ws = ws_ref[...]
    agg = agg_ref[0] + agg_ref[1] + ws * hp_ref[...]
    den = (den_ref[0, 0] + den_ref[0, 1])[:, None] + ws + 1e-16
    return jnp.maximum(agg / den + b_ref[...], 0.0)


def _mid_body(agg_ref, den_ref, ws_ref, hp_ref, b_ref, w_ref, atts_ref,
              attd_ref, h_ref, as_ref, ad_ref, ws2_ref):
    x2 = _norm_x(agg_ref, den_ref, ws_ref, hp_ref, b_ref)
    h = x2 @ w_ref[...]
    a_s = jnp.sum(h * atts_ref[...], axis=1, keepdims=True)
    a_d = jnp.sum(h * attd_ref[...], axis=1, keepdims=True)
    h_ref[...] = h
    as_ref[...] = a_s
    ad_ref[...] = a_d
    ws2_ref[...] = jnp.exp(_leaky(a_s + a_d))


def _tc_mid(agg, den, ws, h_prev, b, W, att_s, att_d):
    return pl.pallas_call(
        _mid_body,
        grid=(N // ROWS,),
        in_specs=[
            pl.BlockSpec((2, ROWS, D), lambda i: (0, i, 0)),
            pl.BlockSpec((1, 2, ROWS), lambda i: (i, 0, 0)),
            pl.BlockSpec((ROWS, 1), lambda i: (i, 0)),
            pl.BlockSpec((ROWS, D), lambda i: (i, 0)),
            pl.BlockSpec((1, D), lambda i: (0, 0)),
            pl.BlockSpec((D, D), lambda i: (0, 0)),
            pl.BlockSpec((1, D), lambda i: (0, 0)),
            pl.BlockSpec((1, D), lambda i: (0, 0)),
        ],
        out_specs=[
            pl.BlockSpec((ROWS, D), lambda i: (i, 0)),
            pl.BlockSpec((ROWS, 1), lambda i: (i, 0)),
            pl.BlockSpec((ROWS, 1), lambda i: (i, 0)),
            pl.BlockSpec((ROWS, 1), lambda i: (i, 0)),
        ],
        out_shape=[
            jax.ShapeDtypeStruct((N, D), jnp.float32),
            jax.ShapeDtypeStruct((N, 1), jnp.float32),
            jax.ShapeDtypeStruct((N, 1), jnp.float32),
            jax.ShapeDtypeStruct((N, 1), jnp.float32),
        ],
    )(agg, den, ws, h_prev, b, W, att_s, att_d)


def _head_body(agg_ref, den_ref, ws_ref, hp_ref, b_ref, fc1w_ref, fc1b_ref,
               fc2w_ref, fc2b_ref, out_ref):
    x3 = _norm_x(agg_ref, den_ref, ws_ref, hp_ref, b_ref)
    z = jnp.maximum(x3 @ fc1w_ref[...] + fc1b_ref[...][None, :], 0.0)
    y = z @ fc2w_ref[...] + fc2b_ref[...][None, :]
    y = y - jnp.max(y, axis=1, keepdims=True)
    e = jnp.exp(y)
    out_ref[...] = e / jnp.sum(e, axis=1, keepdims=True)


def _tc_head(agg, den, ws, h_prev, b, fc1_w, fc1_b, fc2_w, fc2_b):
    return pl.pallas_call(
        _head_body,
        grid=(N // ROWS,),
        in_specs=[
            pl.BlockSpec((2, ROWS, D), lambda i: (0, i, 0)),
            pl.BlockSpec((1, 2, ROWS), lambda i: (i, 0, 0)),
            pl.BlockSpec((ROWS, 1), lambda i: (i, 0)),
            pl.BlockSpec((ROWS, D), lambda i: (i, 0)),
            pl.BlockSpec((1, D), lambda i: (0, 0)),
            pl.BlockSpec((D, D), lambda i: (0, 0)),
            pl.BlockSpec((D,), lambda i: (0,)),
            pl.BlockSpec((D, OUT), lambda i: (0, 0)),
            pl.BlockSpec((OUT,), lambda i: (0,)),
        ],
        out_specs=pl.BlockSpec((ROWS, OUT), lambda i: (i, 0)),
        out_shape=jax.ShapeDtypeStruct((N, OUT), jnp.float32),
    )(agg, den, ws, h_prev, b, fc1_w, fc1_b, fc2_w, fc2_b)


HALF = 64
NT = NCHUNK * 2


def _sc_body(h_hbm, as_hbm, ad_hbm, src_hbm, dst_hbm, agg_out, den_out,
             as_v, ad_v, src_b, dst_b, w_b, rows, agg_sh, den_sh,
             gsem, ssem):
    cid = lax.axis_index("c")
    sid = lax.axis_index("s")
    wid = sid * 2 + cid

    pltpu.sync_copy(as_hbm, as_v)
    pltpu.sync_copy(ad_hbm, ad_v)

    zeros = jnp.zeros((16,), jnp.float32)

    def zero_rows(i, _):
        r = i // (D // 16)
        q = i % (D // 16)
        rows[0, r, pl.ds(q * 16, 16)] = zeros
        return 0
    lax.fori_loop(0, HALF * (D // 16), zero_rows, 0)
    for q in range(16):
        w_b[q // 8, pl.ds((q % 8) * 16, 16)] = zeros

    for q in range(9):
        pltpu.sync_copy(rows.at[0],
                        agg_sh.at[pl.ds(sid * 625 + q * HALF, HALF)])
    pltpu.sync_copy(rows.at[0, pl.ds(0, 625 - 9 * HALF)],
                    agg_sh.at[pl.ds(sid * 625 + 9 * HALF, 625 - 9 * HALF)])
    for q in range(4):
        pltpu.sync_copy(w_b.at[0],
                        den_sh.at[pl.ds(sid * DENW + q * CHUNK, CHUNK)])
    pltpu.sync_copy(w_b.at[0, pl.ds(0, DENW - 512)],
                    den_sh.at[pl.ds(sid * DENW + 512, DENW - 512)])
    plsc.subcore_barrier()

    lane = lax.iota(jnp.int32, 16)

    def load_chunk(c):
        bi = lax.rem(c, 2)
        pltpu.sync_copy(src_hbm.at[wid, c], src_b.at[bi])
        pltpu.sync_copy(dst_hbm.at[wid, c], dst_b.at[bi])
        for hh in range(2):
            for g in range(HALF // 16):
                sv = src_b[bi, hh, pl.ds(g * 16, 16)]
                dv = dst_b[bi, hh, pl.ds(g * 16, 16)]
                e = (plsc.load_gather(as_v, [sv])
                     + plsc.load_gather(ad_v, [dv]))
                w = jnp.exp(_leaky(e))
                valid = (c * CHUNK + hh * HALF + g * 16) + lane < EPW
                w_b[bi, pl.ds(hh * HALF + g * 16, 16)] = \
                    jnp.where(valid, w, 0.0)
        for hh in range(2):
            pltpu.sync_copy(w_b.at[bi, pl.ds(hh * HALF, HALF)],
                            den_sh.at[dst_b.at[bi, hh]], add=True)

    def issue_gather(t):
        c = t // 2
        pltpu.async_copy(
            h_hbm.at[src_b.at[lax.rem(c, 2), lax.rem(t, 2)]],
            rows.at[lax.rem(t, 3)], gsem.at[lax.rem(t, 3)])

    load_chunk(0)
    issue_gather(0)

    def step(t, _):
        c = t // 2
        half = lax.rem(t, 2)
        br = lax.rem(t, 3)
        bi = lax.rem(c, 2)

        @pl.when(t >= 2)
        def _():
            tp = t - 2
            pltpu.make_async_copy(
                rows.at[lax.rem(tp, 3)],
                agg_sh.at[dst_b.at[lax.rem(tp // 2, 2), lax.rem(tp, 2)]],
                ssem.at[lax.rem(tp, 3)]).wait()

        @pl.when((half == 1) & (c + 1 < NCHUNK))
        def _():
            load_chunk(c + 1)

        @pl.when(t + 1 < NT)
        def _():
            issue_gather(t + 1)

        pltpu.make_async_copy(
            h_hbm.at[src_b.at[bi, half]], rows.at[br],
            gsem.at[br]).wait()

        @plsc.parallel_loop(0, HALF, unroll=2)
        def _scale(i):
            wsp = plsc.load_gather(
                w_b, [jnp.full((16,), bi, jnp.int32),
                      jnp.full((16,), half * HALF + i, jnp.int32)])
            for j in range(D // 16):
                rows[br, i, pl.ds(j * 16, 16)] = \
                    rows[br, i, pl.ds(j * 16, 16)] * wsp

        pltpu.async_copy(rows.at[br], agg_sh.at[dst_b.at[bi, half]],
                         ssem.at[br], add=True)
        return 0

    lax.fori_loop(0, NT, step, 0)

    for tp in (NT - 2, NT - 1):
        pltpu.make_async_copy(
            rows.at[tp % 3],
            agg_sh.at[dst_b.at[(tp // 2) % 2, tp % 2]],
            ssem.at[tp % 3]).wait()
    plsc.subcore_barrier()

    for q in range(4):
        pltpu.sync_copy(den_sh.at[pl.ds(sid * DENW + q * CHUNK, CHUNK)],
                        w_b.at[0])
        pltpu.sync_copy(w_b.at[0],
                        den_out.at[pl.ds(cid * EPAD + sid * DENW
                                         + q * CHUNK, CHUNK)])
    pltpu.sync_copy(den_sh.at[pl.ds(sid * DENW + 512, DENW - 512)],
                    w_b.at[0, pl.ds(0, DENW - 512)])
    pltpu.sync_copy(w_b.at[0, pl.ds(0, DENW - 512)],
                    den_out.at[pl.ds(cid * EPAD + sid * DENW + 512,
                                     DENW - 512)])
    pltpu.sync_copy(agg_sh.at[pl.ds(sid * 624, 624)],
                    agg_out.at[cid, pl.ds(sid * 624, 624)])

    @pl.when(sid == NSUB - 1)
    def _tail():
        pltpu.sync_copy(agg_sh.at[pl.ds(9984, 16)],
                        agg_out.at[cid, pl.ds(9984, 16)])


def _sc_layer(h, a_s, a_d, src3, dst3):
    mesh = plsc.VectorSubcoreMesh(core_axis_name="c", subcore_axis_name="s",
                                  num_cores=2, num_subcores=NSUB)
    f = pl.kernel(
        _sc_body,
        out_type=[
            jax.ShapeDtypeStruct((2, N, D), jnp.float32),
            jax.ShapeDtypeStruct((2 * EPAD,), jnp.float32),
        ],
        mesh=mesh,
        compiler_params=pltpu.CompilerParams(needs_layout_passes=False),
        scratch_types=[
            pltpu.VMEM((N,), jnp.float32),
            pltpu.VMEM((N,), jnp.float32),
            pltpu.VMEM((2, 2, HALF), jnp.int32),
            pltpu.VMEM((2, 2, HALF), jnp.int32),
            pltpu.VMEM((2, CHUNK), jnp.float32),
            pltpu.VMEM((3, HALF, D), jnp.float32),
            pltpu.VMEM_SHARED((N, D), jnp.float32),
            pltpu.VMEM_SHARED((EPAD,), jnp.float32),
            pltpu.SemaphoreType.DMA((3,)),
            pltpu.SemaphoreType.DMA((3,)),
        ],
    )
    return f(h, a_s, a_d, src3, dst3)


def kernel(x, edge_index, W1, att_s1, att_d1, b1, W2, att_s2, att_d2, b2,
           fc1_w, fc1_b, fc2_w, fc2_b):
    pad = jnp.zeros((NW, EPAD - EPW), jnp.int32)
    src3 = jnp.concatenate([edge_index[0].reshape(NW, EPW), pad],
                           axis=1).reshape(NW, NCHUNK, 2, HALF)
    dst3 = jnp.concatenate([edge_index[1].reshape(NW, EPW), pad],
                           axis=1).reshape(NW, NCHUNK, 2, HALF)
    b1r = b1.reshape(1, D)
    b2r = b2.reshape(1, D)

    def den_t(d):
        return (d.reshape(2, EPAD)[:, :N]
                .reshape(2, N // ROWS, ROWS).transpose(1, 0, 2))

    h1, as1, ad1, ws1 = _tc_pre(x, W1, att_s1, att_d1)
    agg1, den1 = _sc_layer(h1, as1.reshape(N), ad1.reshape(N), src3, dst3)
    h2, as2, ad2, ws2 = _tc_mid(agg1, den_t(den1), ws1, h1, b1r, W2,
                                att_s2, att_d2)
    agg2, den2 = _sc_layer(h2, as2.reshape(N), ad2.reshape(N), src3, dst3)
    return _tc_head(agg2, den_t(den2), ws2, h2, b2r, fc1_w, fc1_b,
                    fc2_w, fc2_b)


# speedup vs baseline: 31.9035x; 1.2285x over previous
import functools

import jax
import jax.numpy as jnp
from jax import lax
from jax.experimental import pallas as pl
from jax.experimental.pallas import tpu as pltpu
from jax.experimental.pallas import tpu_sc as plsc

N = 10000
D = 128
OUT = 64
ROWS = 1000
NW = 32
EPW = N
CHUNK = 128
NCHUNK = 79
EPAD = NCHUNK * CHUNK
NSUB = 16
DENW = EPAD // NSUB


def _leaky(e):
    return jnp.where(e >= 0.0, e, 0.2 * e)


def _pre_body(x_ref, w_ref, atts_ref, attd_ref, h_ref, as_ref, ad_ref, ws_ref):
    h = x_ref[...] @ w_ref[...]
    a_s = jnp.sum(h * atts_ref[...], axis=1, keepdims=True)
    a_d = jnp.sum(h * attd_ref[...], axis=1, keepdims=True)
    h_ref[...] = h
    as_ref[...] = a_s
    ad_ref[...] = a_d
    ws_ref[...] = jnp.exp(_leaky(a_s + a_d))


def _tc_pre(x, W, att_s, att_d):
    return pl.pallas_call(
        _pre_body,
        grid=(N // ROWS,),
        in_specs=[
            pl.BlockSpec((ROWS, D), lambda i: (i, 0)),
            pl.BlockSpec((D, D), lambda i: (0, 0)),
            pl.BlockSpec((1, D), lambda i: (0, 0)),
            pl.BlockSpec((1, D), lambda i: (0, 0)),
        ],
        out_specs=[
            pl.BlockSpec((ROWS, D), lambda i: (i, 0)),
            pl.BlockSpec((ROWS, 1), lambda i: (i, 0)),
            pl.BlockSpec((ROWS, 1), lambda i: (i, 0)),
            pl.BlockSpec((ROWS, 1), lambda i: (i, 0)),
        ],
        out_shape=[
            jax.ShapeDtypeStruct((N, D), jnp.float32),
            jax.ShapeDtypeStruct((N, 1), jnp.float32),
            jax.ShapeDtypeStruct((N, 1), jnp.float32),
            jax.ShapeDtypeStruct((N, 1), jnp.float32),
        ],
    )(x, W, att_s, att_d)


def _norm_x(agg_ref, den_ref, ws_ref, hp_ref, b_ref):
    ws = ws_ref[...]
    agg = agg_ref[0] + agg_ref[1] + ws * hp_ref[...]
    den = (den_ref[0, 0] + den_ref[0, 1])[:, None] + ws + 1e-16
    return jnp.maximum(agg / den + b_ref[...], 0.0)


def _mid_body(agg_ref, den_ref, ws_ref, hp_ref, b_ref, w_ref, atts_ref,
              attd_ref, h_ref, as_ref, ad_ref, ws2_ref):
    x2 = _norm_x(agg_ref, den_ref, ws_ref, hp_ref, b_ref)
    h = x2 @ w_ref[...]
    a_s = jnp.sum(h * atts_ref[...], axis=1, keepdims=True)
    a_d = jnp.sum(h * attd_ref[...], axis=1, keepdims=True)
    h_ref[...] = h
    as_ref[...] = a_s
    ad_ref[...] = a_d
    ws2_ref[...] = jnp.exp(_leaky(a_s + a_d))


def _tc_mid(agg, den, ws, h_prev, b, W, att_s, att_d):
    return pl.pallas_call(
        _mid_body,
        grid=(N // ROWS,),
        in_specs=[
            pl.BlockSpec((2, ROWS, D), lambda i: (0, i, 0)),
            pl.BlockSpec((1, 2, ROWS), lambda i: (i, 0, 0)),
            pl.BlockSpec((ROWS, 1), lambda i: (i, 0)),
            pl.BlockSpec((ROWS, D), lambda i: (i, 0)),
            pl.BlockSpec((1, D), lambda i: (0, 0)),
            pl.BlockSpec((D, D), lambda i: (0, 0)),
            pl.BlockSpec((1, D), lambda i: (0, 0)),
            pl.BlockSpec((1, D), lambda i: (0, 0)),
        ],
        out_specs=[
            pl.BlockSpec((ROWS, D), lambda i: (i, 0)),
            pl.BlockSpec((ROWS, 1), lambda i: (i, 0)),
            pl.BlockSpec((ROWS, 1), lambda i: (i, 0)),
            pl.BlockSpec((ROWS, 1), lambda i: (i, 0)),
        ],
        out_shape=[
            jax.ShapeDtypeStruct((N, D), jnp.float32),
            jax.ShapeDtypeStruct((N, 1), jnp.float32),
            jax.ShapeDtypeStruct((N, 1), jnp.float32),
            jax.ShapeDtypeStruct((N, 1), jnp.float32),
        ],
    )(agg, den, ws, h_prev, b, W, att_s, att_d)


def _head_body(agg_ref, den_ref, ws_ref, hp_ref, b_ref, fc1w_ref, fc1b_ref,
               fc2w_ref, fc2b_ref, out_ref):
    x3 = _norm_x(agg_ref, den_ref, ws_ref, hp_ref, b_ref)
    z = jnp.maximum(x3 @ fc1w_ref[...] + fc1b_ref[...][None, :], 0.0)
    y = z @ fc2w_ref[...] + fc2b_ref[...][None, :]
    y = y - jnp.max(y, axis=1, keepdims=True)
    e = jnp.exp(y)
    out_ref[...] = e / jnp.sum(e, axis=1, keepdims=True)


def _tc_head(agg, den, ws, h_prev, b, fc1_w, fc1_b, fc2_w, fc2_b):
    return pl.pallas_call(
        _head_body,
        grid=(N // ROWS,),
        in_specs=[
            pl.BlockSpec((2, ROWS, D), lambda i: (0, i, 0)),
            pl.BlockSpec((1, 2, ROWS), lambda i: (i, 0, 0)),
            pl.BlockSpec((ROWS, 1), lambda i: (i, 0)),
            pl.BlockSpec((ROWS, D), lambda i: (i, 0)),
            pl.BlockSpec((1, D), lambda i: (0, 0)),
            pl.BlockSpec((D, D), lambda i: (0, 0)),
            pl.BlockSpec((D,), lambda i: (0,)),
            pl.BlockSpec((D, OUT), lambda i: (0, 0)),
            pl.BlockSpec((OUT,), lambda i: (0,)),
        ],
        out_specs=pl.BlockSpec((ROWS, OUT), lambda i: (i, 0)),
        out_shape=jax.ShapeDtypeStruct((N, OUT), jnp.float32),
    )(agg, den, ws, h_prev, b, fc1_w, fc1_b, fc2_w, fc2_b)


HALF = 64
NT = NCHUNK * 2


def _sc_body(h_hbm, as_hbm, ad_hbm, src_hbm, dst_hbm, agg_out, den_out,
             as_v, ad_v, src_b, dst_b, w_b, rows, agg_sh, den_sh,
             gsem, ssem, isem, dsem):
    cid = lax.axis_index("c")
    sid = lax.axis_index("s")
    wid = sid * 2 + cid

    pltpu.sync_copy(as_hbm, as_v)
    pltpu.sync_copy(ad_hbm, ad_v)

    zeros = jnp.zeros((16,), jnp.float32)

    def zero_rows(i, _):
        r = i // (D // 16)
        q = i % (D // 16)
        rows[0, r, pl.ds(q * 16, 16)] = zeros
        return 0
    lax.fori_loop(0, HALF * (D // 16), zero_rows, 0)
    for q in range(8):
        w_b[0, pl.ds(q * 16, 16)] = zeros

    for q in range(9):
        pltpu.sync_copy(rows.at[0],
                        agg_sh.at[pl.ds(sid * 625 + q * HALF, HALF)])
    pltpu.sync_copy(rows.at[0, pl.ds(0, 625 - 9 * HALF)],
                    agg_sh.at[pl.ds(sid * 625 + 9 * HALF, 625 - 9 * HALF)])
    for q in range(4):
        pltpu.sync_copy(w_b.at[0],
                        den_sh.at[pl.ds(sid * DENW + q * CHUNK, CHUNK)])
    pltpu.sync_copy(w_b.at[0, pl.ds(0, DENW - 512)],
                    den_sh.at[pl.ds(sid * DENW + 512, DENW - 512)])
    plsc.subcore_barrier()

    lane = lax.iota(jnp.int32, 16)

    def issue_idx(c):
        ci = lax.rem(c, 3)
        pltpu.async_copy(src_hbm.at[wid, c], src_b.at[ci], isem.at[ci])
        pltpu.async_copy(dst_hbm.at[wid, c], dst_b.at[ci], isem.at[ci])

    def wait_idx(c):
        ci = lax.rem(c, 3)
        pltpu.make_async_copy(src_hbm.at[wid, c], src_b.at[ci],
                              isem.at[ci]).wait()
        pltpu.make_async_copy(dst_hbm.at[wid, c], dst_b.at[ci],
                              isem.at[ci]).wait()

    def scalar_phase(c):
        ci = lax.rem(c, 3)
        for hh in range(2):
            for g in range(HALF // 16):
                sv = src_b[ci, hh, pl.ds(g * 16, 16)]
                dv = dst_b[ci, hh, pl.ds(g * 16, 16)]
                e = (plsc.load_gather(as_v, [sv])
                     + plsc.load_gather(ad_v, [dv]))
                w = jnp.exp(_leaky(e))
                valid = (c * CHUNK + hh * HALF + g * 16) + lane < EPW
                w_b[ci, pl.ds(hh * HALF + g * 16, 16)] = \
                    jnp.where(valid, w, 0.0)
        for hh in range(2):
            pltpu.async_copy(w_b.at[ci, pl.ds(hh * HALF, HALF)],
                             den_sh.at[dst_b.at[ci, hh]], dsem.at[ci],
                             add=True)

    def wait_den(c):
        ci = lax.rem(c, 3)
        for hh in range(2):
            pltpu.make_async_copy(w_b.at[ci, pl.ds(hh * HALF, HALF)],
                                  den_sh.at[dst_b.at[ci, hh]],
                                  dsem.at[ci]).wait()

    def issue_gather(t):
        c = t // 2
        pltpu.async_copy(
            h_hbm.at[src_b.at[lax.rem(c, 3), lax.rem(t, 2)]],
            rows.at[lax.rem(t, 3)], gsem.at[lax.rem(t, 3)])

    issue_idx(0)
    issue_idx(1)
    wait_idx(0)
    scalar_phase(0)
    wait_idx(1)
    scalar_phase(1)
    issue_gather(0)
    issue_gather(1)

    def step(t, _):
        c = t // 2
        half = lax.rem(t, 2)
        br = lax.rem(t, 3)

        @pl.when(t >= 2)
        def _():
            tp = t - 2
            pltpu.make_async_copy(
                rows.at[lax.rem(tp, 3)],
                agg_sh.at[dst_b.at[lax.rem(lax.div(tp, 2), 3),
                                   lax.rem(tp, 2)]],
                ssem.at[lax.rem(tp, 3)]).wait()

        @pl.when((half == 0) & (c + 2 < NCHUNK))
        def _():
            @pl.when(c >= 1)
            def _():
                wait_den(c - 1)
            issue_idx(c + 2)

        @pl.when((half == 1) & (c + 1 >= 2) & (c + 1 < NCHUNK))
        def _():
            wait_idx(c + 1)
            scalar_phase(c + 1)

        @pl.when((t + 1 >= 2) & (t + 1 < NT))
        def _():
            issue_gather(t + 1)

        pltpu.make_async_copy(
            h_hbm.at[src_b.at[lax.rem(c, 3), half]], rows.at[br],
            gsem.at[br]).wait()

        @plsc.parallel_loop(0, HALF, unroll=2)
        def _scale(i):
            wsp = plsc.load_gather(
                w_b, [jnp.full((16,), lax.rem(c, 3), jnp.int32),
                      jnp.full((16,), half * HALF + i, jnp.int32)])
            for j in range(D // 16):
                rows[br, i, pl.ds(j * 16, 16)] = \
                    rows[br, i, pl.ds(j * 16, 16)] * wsp

        pltpu.async_copy(rows.at[br],
                         agg_sh.at[dst_b.at[lax.rem(c, 3), half]],
                         ssem.at[br], add=True)
        return 0

    lax.fori_loop(0, NT, step, 0)

    for tp in (NT - 2, NT - 1):
        pltpu.make_async_copy(
            rows.at[tp % 3],
            agg_sh.at[dst_b.at[(tp // 2) % 3, tp % 2]],
            ssem.at[tp % 3]).wait()
    for cp in (NCHUNK - 3, NCHUNK - 2, NCHUNK - 1):
        wait_den(cp)
    plsc.subcore_barrier()

    for q in range(4):
        pltpu.sync_copy(den_sh.at[pl.ds(sid * DENW + q * CHUNK, CHUNK)],
                        w_b.at[0])
        pltpu.sync_copy(w_b.at[0],
                        den_out.at[pl.ds(cid * EPAD + sid * DENW
                                         + q * CHUNK, CHUNK)])
    pltpu.sync_copy(den_sh.at[pl.ds(sid * DENW + 512, DENW - 512)],
                    w_b.at[0, pl.ds(0, DENW - 512)])
    pltpu.sync_copy(w_b.at[0, pl.ds(0, DENW - 512)],
                    den_out.at[pl.ds(cid * EPAD + sid * DENW + 512,
                                     DENW - 512)])
    pltpu.sync_copy(agg_sh.at[pl.ds(sid * 624, 624)],
                    agg_out.at[cid, pl.ds(sid * 624, 624)])

    @pl.when(sid == NSUB - 1)
    def _tail():
        pltpu.sync_copy(agg_sh.at[pl.ds(9984, 16)],
                        agg_out.at[cid, pl.ds(9984, 16)])


def _sc_layer(h, a_s, a_d, src3, dst3):
    mesh = plsc.VectorSubcoreMesh(core_axis_name="c", subcore_axis_name="s",
                                  num_cores=2, num_subcores=NSUB)
    f = pl.kernel(
        _sc_body,
        out_type=[
            jax.ShapeDtypeStruct((2, N, D), jnp.float32),
            jax.ShapeDtypeStruct((2 * EPAD,), jnp.float32),
        ],
        mesh=mesh,
        compiler_params=pltpu.CompilerParams(needs_layout_passes=False),
        scratch_types=[
            pltpu.VMEM((N,), jnp.float32),
            pltpu.VMEM((N,), jnp.float32),
            pltpu.VMEM((3, 2, HALF), jnp.int32),
            pltpu.VMEM((3, 2, HALF), jnp.int32),
            pltpu.VMEM((3, CHUNK), jnp.float32),
            pltpu.VMEM((3, HALF, D), jnp.float32),
            pltpu.VMEM_SHARED((N, D), jnp.float32),
            pltpu.VMEM_SHARED((EPAD,), jnp.float32),
            pltpu.SemaphoreType.DMA((3,)),
            pltpu.SemaphoreType.DMA((3,)),
            pltpu.SemaphoreType.DMA((3,)),
            pltpu.SemaphoreType.DMA((3,)),
        ],
    )
    return f(h, a_s, a_d, src3, dst3)


def kernel(x, edge_index, W1, att_s1, att_d1, b1, W2, att_s2, att_d2, b2,
           fc1_w, fc1_b, fc2_w, fc2_b):
    pad = jnp.zeros((NW, EPAD - EPW), jnp.int32)
    src3 = jnp.concatenate([edge_index[0].reshape(NW, EPW), pad],
                           axis=1).reshape(NW, NCHUNK, 2, HALF)
    dst3 = jnp.concatenate([edge_index[1].reshape(NW, EPW), pad],
                           axis=1).reshape(NW, NCHUNK, 2, HALF)
    b1r = b1.reshape(1, D)
    b2r = b2.reshape(1, D)

    def den_t(d):
        return (d.reshape(2, EPAD)[:, :N]
                .reshape(2, N // ROWS, ROWS).transpose(1, 0, 2))

    h1, as1, ad1, ws1 = _tc_pre(x, W1, att_s1, att_d1)
    agg1, den1 = _sc_layer(h1, as1.reshape(N), ad1.reshape(N), src3, dst3)
    h2, as2, ad2, ws2 = _tc_mid(agg1, den_t(den1), ws1, h1, b1r, W2,
                                att_s2, att_d2)
    agg2, den2 = _sc_layer(h2, as2.reshape(N), ad2.reshape(N), src3, dst3)
    return _tc_head(agg2, den_t(den2), ws2, h2, b2r, fc1_w, fc1_b,
                    fc2_w, fc2_b)
